# baseline (speedup 1.0000x reference)
import functools

import jax
import jax.numpy as jnp
from jax.experimental import pallas as pl
from jax.experimental.pallas import tpu as pltpu


def _router_body(x_ref, wg_ref, tope_ref, topw_ref):
    x = x_ref[...]
    logits = jax.lax.dot_general(
        x, wg_ref[...], (((1,), (1,)), ((), ())),
        preferred_element_type=jnp.float32)
    m = jnp.max(logits, axis=1, keepdims=True)
    p = jnp.exp(logits - m)
    ne = p.shape[1]
    idx = jax.lax.broadcasted_iota(jnp.int32, p.shape, 1)
    m1 = jnp.max(p, axis=1, keepdims=True)
    i1 = jnp.min(jnp.where(p == m1, idx, ne), axis=1, keepdims=True)
    p2 = jnp.where(idx == i1, -jnp.inf, p)
    m2 = jnp.max(p2, axis=1, keepdims=True)
    i2 = jnp.min(jnp.where(p2 == m2, idx, ne), axis=1, keepdims=True)
    wsum = m1 + m2
    tope_ref[...] = jnp.concatenate([i1, i2], axis=1)
    topw_ref[...] = jnp.concatenate([m1 / wsum, m2 / wsum], axis=1)


def _gateup_body(s_ref, x_ref, wg_ref, wu_ref, h_ref):
    x = x_ref[...]
    g = jax.lax.dot_general(x, wg_ref[0].astype(jnp.bfloat16),
                            (((1,), (1,)), ((), ())),
                            preferred_element_type=jnp.float32)
    u = jax.lax.dot_general(x, wu_ref[0].astype(jnp.bfloat16),
                            (((1,), (1,)), ((), ())),
                            preferred_element_type=jnp.float32)
    h_ref[...] = ((g * jax.lax.logistic(g)) * u).astype(jnp.bfloat16)


def _down_body(s_ref, h_ref, wd_ref, y_ref):
    h = h_ref[...]
    y = jax.lax.dot_general(h, wd_ref[0].astype(jnp.bfloat16),
                            (((1,), (1,)), ((), ())),
                            preferred_element_type=jnp.float32)
    y_ref[...] = y.astype(jnp.bfloat16)


def kernel(hidden_states, Wg, Wgate, Wup, Wdown):
    B, S, H = hidden_states.shape
    E, I, _ = Wgate.shape
    T = B * S
    K = 2
    TM = 256
    P = T * K + E * TM
    NT = P // TM
    flat = hidden_states.reshape(T, H)

    n_rt = 2 if T % 2 == 0 else 1
    TR = T // n_rt
    tope, topw = pl.pallas_call(
        _router_body,
        grid=(n_rt,),
        in_specs=[
            pl.BlockSpec((TR, H), lambda t: (t, 0)),
            pl.BlockSpec((E, H), lambda t: (0, 0)),
        ],
        out_specs=[
            pl.BlockSpec((TR, K), lambda t: (t, 0)),
            pl.BlockSpec((TR, K), lambda t: (t, 0)),
        ],
        out_shape=[
            jax.ShapeDtypeStruct((T, K), jnp.int32),
            jax.ShapeDtypeStruct((T, K), jnp.float32),
        ],
    )(flat, Wg)

    e_all = tope.T.reshape(T * K)
    w_all = topw.T.reshape(T * K)
    tok_all = jnp.tile(jnp.arange(T, dtype=jnp.int32), K)
    onehot = (e_all[:, None] == jnp.arange(E, dtype=jnp.int32)[None, :])
    pref = jnp.cumsum(onehot.astype(jnp.int32), axis=0)
    counts = pref[-1]
    rank = jnp.take_along_axis(pref - onehot.astype(jnp.int32),
                               e_all[:, None], axis=1)[:, 0]
    starts = [jnp.int32(0)]
    for e in range(1, E):
        nxt = starts[-1] + counts[e - 1]
        starts.append(((nxt + TM - 1) // TM) * TM)
    aligned_start = jnp.stack(starts)
    dest = aligned_start[e_all] + rank
    row_token = jnp.zeros((P,), jnp.int32).at[dest].set(
        tok_all, unique_indices=True)
    tile_expert = jnp.sum(
        (jnp.arange(NT, dtype=jnp.int32)[:, None] * TM
         >= aligned_start[None, :]).astype(jnp.int32), axis=1) - 1
    pos0, pos1 = dest[:T], dest[T:]

    x_sorted = jnp.take(flat.astype(jnp.bfloat16), row_token, axis=0)

    iblk = 1408 if I % 1408 == 0 else I
    n_i = I // iblk
    h_sorted = pl.pallas_call(
        _gateup_body,
        grid_spec=pltpu.PrefetchScalarGridSpec(
            num_scalar_prefetch=1,
            grid=(n_i, NT),
            in_specs=[
                pl.BlockSpec((TM, H), lambda i, t, s: (t, 0)),
                pl.BlockSpec((1, iblk, H), lambda i, t, s: (s[t], i, 0)),
                pl.BlockSpec((1, iblk, H), lambda i, t, s: (s[t], i, 0)),
            ],
            out_specs=pl.BlockSpec((TM, iblk), lambda i, t, s: (t, i)),
        ),
        out_shape=jax.ShapeDtypeStruct((P, I), jnp.bfloat16),
    )(tile_expert, x_sorted, Wgate, Wup)

    y_sorted = pl.pallas_call(
        _down_body,
        grid_spec=pltpu.PrefetchScalarGridSpec(
            num_scalar_prefetch=1,
            grid=(NT,),
            in_specs=[
                pl.BlockSpec((TM, I), lambda t, s: (t, 0)),
                pl.BlockSpec((1, H, I), lambda t, s: (s[t], 0, 0)),
            ],
            out_specs=pl.BlockSpec((TM, H), lambda t, s: (t, 0)),
        ),
        out_shape=jax.ShapeDtypeStruct((P, H), jnp.bfloat16),
    )(tile_expert, h_sorted, Wdown)

    w0, w1 = topw[:, 0:1], topw[:, 1:2]
    out = (w0 * jnp.take(y_sorted, pos0, axis=0)
           + w1 * jnp.take(y_sorted, pos1, axis=0))
    return out.reshape(B, S, H)


# speedup vs baseline: 1.0589x; 1.0589x over previous
import functools

import jax
import jax.numpy as jnp
from jax import lax
from jax.experimental import pallas as pl
from jax.experimental.pallas import tpu as pltpu
from jax.experimental.pallas import tpu_sc as plsc


def _router_body(x_ref, wg_ref, tope_ref, topw_ref):
    x = x_ref[...]
    logits = jax.lax.dot_general(
        x, wg_ref[...], (((1,), (1,)), ((), ())),
        preferred_element_type=jnp.float32)
    m = jnp.max(logits, axis=1, keepdims=True)
    p = jnp.exp(logits - m)
    ne = p.shape[1]
    idx = jax.lax.broadcasted_iota(jnp.int32, p.shape, 1)
    m1 = jnp.max(p, axis=1, keepdims=True)
    i1 = jnp.min(jnp.where(p == m1, idx, ne), axis=1, keepdims=True)
    p2 = jnp.where(idx == i1, -jnp.inf, p)
    m2 = jnp.max(p2, axis=1, keepdims=True)
    i2 = jnp.min(jnp.where(p2 == m2, idx, ne), axis=1, keepdims=True)
    wsum = m1 + m2
    tope_ref[...] = jnp.concatenate([i1, i2], axis=1)
    topw_ref[...] = jnp.concatenate([m1 / wsum, m2 / wsum], axis=1)


def _gateup_body(s_ref, x_ref, wg_ref, wu_ref, h_ref):
    x = x_ref[...]
    g = jax.lax.dot_general(x, wg_ref[0].astype(jnp.bfloat16),
                            (((1,), (1,)), ((), ())),
                            preferred_element_type=jnp.float32)
    u = jax.lax.dot_general(x, wu_ref[0].astype(jnp.bfloat16),
                            (((1,), (1,)), ((), ())),
                            preferred_element_type=jnp.float32)
    h_ref[...] = ((g * jax.lax.logistic(g)) * u).astype(jnp.bfloat16)


def _down_body(s_ref, h_ref, wd_ref, rw_ref, y_ref):
    h = h_ref[...]
    y = jax.lax.dot_general(h, wd_ref[0].astype(jnp.bfloat16),
                            (((1,), (1,)), ((), ())),
                            preferred_element_type=jnp.float32)
    y_ref[...] = y * rw_ref[...]


def _combine_sc(y_sorted, dest, T, H):
    info = plsc.get_sparse_core_info()
    NW = info.num_cores * info.num_subcores
    per_w = T // NW
    CH = 32
    n_sub = per_w // CH
    mesh = plsc.VectorSubcoreMesh(core_axis_name="c", subcore_axis_name="s")

    @functools.partial(
        pl.kernel, mesh=mesh,
        out_type=jax.ShapeDtypeStruct((T, H), jnp.float32),
        scratch_types=[
            pltpu.VMEM((CH,), jnp.int32),
            pltpu.VMEM((CH,), jnp.int32),
            pltpu.VMEM((CH, H), jnp.float32),
            pltpu.VMEM((CH, H), jnp.float32),
            pltpu.SemaphoreType.DMA,
            pltpu.SemaphoreType.DMA,
        ],
    )
    def k(y_hbm, dest_hbm, out_hbm, idx0_v, idx1_v, rows0_v, rows1_v,
          sem0, sem1):
        wid = lax.axis_index("s") * info.num_cores + lax.axis_index("c")
        for sub in range(n_sub):
            base = wid * per_w + sub * CH
            pltpu.sync_copy(dest_hbm.at[pl.ds(base, CH)], idx0_v)
            pltpu.sync_copy(dest_hbm.at[pl.ds(T + base, CH)], idx1_v)
            cp0 = pltpu.async_copy(y_hbm.at[idx0_v], rows0_v, sem0)
            cp1 = pltpu.async_copy(y_hbm.at[idx1_v], rows1_v, sem1)
            cp0.wait()
            cp1.wait()

            def row_add(r, carry):
                r0 = rows0_v.at[r]
                r1 = rows1_v.at[r]
                for c in range(H // 16):
                    sl = pl.ds(c * 16, 16)
                    r0[sl] = r0[sl] + r1[sl]
                return carry

            lax.fori_loop(0, CH, row_add, 0)
            pltpu.sync_copy(rows0_v, out_hbm.at[pl.ds(base, CH)])

    return k(y_sorted, dest)


def kernel(hidden_states, Wg, Wgate, Wup, Wdown):
    B, S, H = hidden_states.shape
    E, I, _ = Wgate.shape
    T = B * S
    K = 2
    TM = 256
    P = T * K + E * TM
    NT = P // TM
    flat = hidden_states.reshape(T, H)

    n_rt = 2 if T % 2 == 0 else 1
    TR = T // n_rt
    tope, topw = pl.pallas_call(
        _router_body,
        grid=(n_rt,),
        in_specs=[
            pl.BlockSpec((TR, H), lambda t: (t, 0)),
            pl.BlockSpec((E, H), lambda t: (0, 0)),
        ],
        out_specs=[
            pl.BlockSpec((TR, K), lambda t: (t, 0)),
            pl.BlockSpec((TR, K), lambda t: (t, 0)),
        ],
        out_shape=[
            jax.ShapeDtypeStruct((T, K), jnp.int32),
            jax.ShapeDtypeStruct((T, K), jnp.float32),
        ],
    )(flat, Wg)

    e_all = tope.T.reshape(T * K)
    w_all = topw.T.reshape(T * K)
    tok_all = jnp.tile(jnp.arange(T, dtype=jnp.int32), K)
    onehot = (e_all[:, None] == jnp.arange(E, dtype=jnp.int32)[None, :])
    pref = jnp.cumsum(onehot.astype(jnp.int32), axis=0)
    counts = pref[-1]
    rank = jnp.take_along_axis(pref - onehot.astype(jnp.int32),
                               e_all[:, None], axis=1)[:, 0]
    starts = [jnp.int32(0)]
    for e in range(1, E):
        nxt = starts[-1] + counts[e - 1]
        starts.append(((nxt + TM - 1) // TM) * TM)
    aligned_start = jnp.stack(starts)
    dest = aligned_start[e_all] + rank
    row_token = jnp.zeros((P,), jnp.int32).at[dest].set(
        tok_all, unique_indices=True)
    row_weight = jnp.zeros((P,), jnp.float32).at[dest].set(
        w_all, unique_indices=True)
    tile_expert = jnp.sum(
        (jnp.arange(NT, dtype=jnp.int32)[:, None] * TM
         >= aligned_start[None, :]).astype(jnp.int32), axis=1) - 1
    pos0, pos1 = dest[:T], dest[T:]

    x_sorted = jnp.take(flat.astype(jnp.bfloat16), row_token, axis=0)

    h_sorted = pl.pallas_call(
        _gateup_body,
        grid_spec=pltpu.PrefetchScalarGridSpec(
            num_scalar_prefetch=1,
            grid=(NT,),
            in_specs=[
                pl.BlockSpec((TM, H), lambda t, s: (t, 0)),
                pl.BlockSpec((1, I, H), lambda t, s: (s[t], 0, 0)),
                pl.BlockSpec((1, I, H), lambda t, s: (s[t], 0, 0)),
            ],
            out_specs=pl.BlockSpec((TM, I), lambda t, s: (t, 0)),
        ),
        out_shape=jax.ShapeDtypeStruct((P, I), jnp.bfloat16),
    )(tile_expert, x_sorted, Wgate, Wup)

    y_sorted = pl.pallas_call(
        _down_body,
        grid_spec=pltpu.PrefetchScalarGridSpec(
            num_scalar_prefetch=1,
            grid=(NT,),
            in_specs=[
                pl.BlockSpec((TM, I), lambda t, s: (t, 0)),
                pl.BlockSpec((1, H, I), lambda t, s: (s[t], 0, 0)),
                pl.BlockSpec((TM, 1), lambda t, s: (t, 0)),
            ],
            out_specs=pl.BlockSpec((TM, H), lambda t, s: (t, 0)),
        ),
        out_shape=jax.ShapeDtypeStruct((P, H), jnp.float32),
    )(tile_expert, h_sorted, Wdown, row_weight.reshape(P, 1))

    out = _combine_sc(y_sorted, dest, T, H)
    return out.reshape(B, S, H)
